# Initial kernel scaffold; baseline (speedup 1.0000x reference)
import functools

import jax
import jax.numpy as jnp
from jax import lax
from jax.experimental import pallas as pl
from jax.experimental.pallas import tpu as pltpu
from jax.experimental.pallas import tpu_sc as plsc

B = 4096
INPUT_DIM = 2048
NUM_LATENTS = 64
EMBED_DIM = 64
NUM_EMB = 1024
CC = 0.25
HID = 64
N_FLAT = B * NUM_LATENTS

BB = 512
GRID = B // BB

SC_NC = 2
SC_NS = 16
SC_NW = SC_NC * SC_NS
SC_PER_W = N_FLAT // SC_NW
SC_CH = 1024


def _main_body(x_ref, w1_ref, b1_ref, w2s_ref, b2r_ref, cbt_ref, c2_ref,
               idx_ref, loss_ref):
    h = jnp.dot(x_ref[...], w1_ref[...], preferred_element_type=jnp.float32)
    h = jnp.maximum(h + b1_ref[...], 0.0)

    total = jnp.float32(0.0)
    idx_acc = jnp.zeros((BB, NUM_LATENTS), dtype=jnp.int32)
    lane = lax.broadcasted_iota(jnp.int32, (BB, NUM_LATENTS), 1)
    for i in range(NUM_LATENTS):
        z = jnp.dot(h, w2s_ref[i], preferred_element_type=jnp.float32)
        z = z + b2r_ref[i:i + 1, :]
        f2 = jnp.sum(z * z, axis=1, keepdims=True)
        mm = jnp.dot(z, cbt_ref[...], preferred_element_type=jnp.float32)
        d = (f2 + c2_ref[...]) - 2.0 * mm
        ai = jnp.argmin(d, axis=1).astype(jnp.int32)[:, None]
        total = total + jnp.sum(jnp.min(d, axis=1))
        idx_acc = jnp.where(lane == i, ai, idx_acc)
    idx_ref[...] = idx_acc

    @pl.when(pl.program_id(0) == 0)
    def _init():
        loss_ref[0, 0] = total

    @pl.when(pl.program_id(0) != 0)
    def _acc():
        loss_ref[0, 0] += total


def _run_main(x, enc_w1, enc_b1, w2s, b2r, cbt, c2):
    return pl.pallas_call(
        _main_body,
        grid=(GRID,),
        in_specs=[
            pl.BlockSpec((BB, INPUT_DIM), lambda b: (b, 0)),
            pl.BlockSpec((INPUT_DIM, HID), lambda b: (0, 0)),
            pl.BlockSpec((1, HID), lambda b: (0, 0)),
            pl.BlockSpec((NUM_LATENTS, HID, EMBED_DIM), lambda b: (0, 0, 0)),
            pl.BlockSpec((NUM_LATENTS, EMBED_DIM), lambda b: (0, 0)),
            pl.BlockSpec((EMBED_DIM, NUM_EMB), lambda b: (0, 0)),
            pl.BlockSpec((1, NUM_EMB), lambda b: (0, 0)),
        ],
        out_specs=[
            pl.BlockSpec((BB, NUM_LATENTS), lambda b: (b, 0)),
            pl.BlockSpec(block_shape=(1, 1), index_map=lambda b: (0, 0),
                         memory_space=pltpu.SMEM),
        ],
        out_shape=[
            jax.ShapeDtypeStruct((B, NUM_LATENTS), jnp.int32),
            jax.ShapeDtypeStruct((1, 1), jnp.float32),
        ],
    )(x, enc_w1, enc_b1, w2s, b2r, cbt, c2)


def _sc_gather(codebook, idx_flat):
    mesh = plsc.VectorSubcoreMesh(core_axis_name="c", subcore_axis_name="s")

    @functools.partial(
        pl.kernel, mesh=mesh,
        out_type=jax.ShapeDtypeStruct((N_FLAT, EMBED_DIM), jnp.float32),
        scratch_types=[
            pltpu.VMEM((SC_CH,), jnp.int32),
            pltpu.VMEM((SC_CH, EMBED_DIM), jnp.float32),
            pltpu.SemaphoreType.DMA,
        ],
    )
    def k(idx_hbm, table_hbm, out_hbm, idx_v, rows_v, sem):
        wid = lax.axis_index("s") * SC_NC + lax.axis_index("c")
        for chunk in range(SC_PER_W // SC_CH):
            base = wid * SC_PER_W + chunk * SC_CH
            pltpu.sync_copy(idx_hbm.at[pl.ds(base, SC_CH)], idx_v)
            pltpu.async_copy(table_hbm.at[idx_v], rows_v, sem).wait()
            pltpu.sync_copy(rows_v, out_hbm.at[pl.ds(base, SC_CH)])

    return k(idx_flat, codebook)


def _dec_body(zq_ref, w1_ref, b1_ref, w2_ref, b2_ref, out_ref):
    hd = jnp.dot(zq_ref[...], w1_ref[...], preferred_element_type=jnp.float32)
    hd = jnp.maximum(hd + b1_ref[...], 0.0)
    out_ref[...] = jnp.dot(hd, w2_ref[...],
                           preferred_element_type=jnp.float32) + b2_ref[...]


def _run_dec(zq, dec_w1, dec_b1, dec_w2, dec_b2):
    return pl.pallas_call(
        _dec_body,
        grid=(GRID,),
        in_specs=[
            pl.BlockSpec((BB, NUM_LATENTS * EMBED_DIM), lambda b: (b, 0)),
            pl.BlockSpec((NUM_LATENTS * EMBED_DIM, HID), lambda b: (0, 0)),
            pl.BlockSpec((1, HID), lambda b: (0, 0)),
            pl.BlockSpec((HID, INPUT_DIM), lambda b: (0, 0)),
            pl.BlockSpec((1, INPUT_DIM), lambda b: (0, 0)),
        ],
        out_specs=pl.BlockSpec((BB, INPUT_DIM), lambda b: (b, 0)),
        out_shape=jax.ShapeDtypeStruct((B, INPUT_DIM), jnp.float32),
    )(zq, dec_w1, dec_b1, dec_w2, dec_b2)


def kernel(x, enc_w1, enc_b1, enc_w2, enc_b2, codebook,
           dec_w1, dec_b1, dec_w2, dec_b2):
    x = x.astype(jnp.float32)
    w2s = enc_w2.reshape(HID, NUM_LATENTS, EMBED_DIM).transpose(1, 0, 2)
    b2r = enc_b2.reshape(NUM_LATENTS, EMBED_DIM)
    cbt = codebook.T
    c2 = jnp.sum(codebook ** 2, axis=1)[None, :]

    idx, loss_sum = _run_main(x, enc_w1, enc_b1[None, :], w2s, b2r, cbt, c2)

    idx_flat = idx.reshape(N_FLAT)
    zq_flat = _sc_gather(codebook, idx_flat)
    zq = zq_flat.reshape(B, NUM_LATENTS * EMBED_DIM)

    x_recon = _run_dec(zq, dec_w1, dec_b1[None, :], dec_w2, dec_b2[None, :])

    vq_loss = (1.0 + CC) * loss_sum[0, 0] / jnp.float32(N_FLAT * EMBED_DIM)

    avg_probs = jnp.float32(1.0 / NUM_EMB)
    perplexity = jnp.exp(-(avg_probs * jnp.log(avg_probs + 1e-10)))

    encoding_indices = idx.reshape(B, NUM_LATENTS, 1)
    return (x_recon, vq_loss, perplexity, encoding_indices)


# speedup vs baseline: 2.1630x; 2.1630x over previous
import functools

import jax
import jax.numpy as jnp
from jax import lax
from jax.experimental import pallas as pl
from jax.experimental.pallas import tpu as pltpu
from jax.experimental.pallas import tpu_sc as plsc

B = 4096
INPUT_DIM = 2048
NUM_LATENTS = 64
EMBED_DIM = 64
NUM_EMB = 1024
CC = 0.25
HID = 64
N_FLAT = B * NUM_LATENTS

BB = 512
GRID = B // BB
FB = 2048
FGRID = N_FLAT // FB

SC_NC = 2
SC_NS = 16
SC_NW = SC_NC * SC_NS
SC_PER_W = N_FLAT // SC_NW
SC_CH = 1024


def _enc_body(x_ref, w1_ref, b1_ref, w2_ref, b2_ref, zf_ref):
    h = jnp.dot(x_ref[...], w1_ref[...], preferred_element_type=jnp.float32)
    h = jnp.maximum(h + b1_ref[...], 0.0)
    ze = jnp.dot(h, w2_ref[...], preferred_element_type=jnp.float32)
    zf_ref[...] = ze + b2_ref[...]


def _run_enc(x, enc_w1, enc_b1, enc_w2, enc_b2):
    return pl.pallas_call(
        _enc_body,
        grid=(GRID,),
        in_specs=[
            pl.BlockSpec((BB, INPUT_DIM), lambda b: (b, 0)),
            pl.BlockSpec((INPUT_DIM, HID), lambda b: (0, 0)),
            pl.BlockSpec((1, HID), lambda b: (0, 0)),
            pl.BlockSpec((HID, NUM_LATENTS * EMBED_DIM), lambda b: (0, 0)),
            pl.BlockSpec((1, NUM_LATENTS * EMBED_DIM), lambda b: (0, 0)),
        ],
        out_specs=pl.BlockSpec((BB, NUM_LATENTS * EMBED_DIM), lambda b: (b, 0)),
        out_shape=jax.ShapeDtypeStruct((B, NUM_LATENTS * EMBED_DIM),
                                       jnp.float32),
    )(x, enc_w1, enc_b1, enc_w2, enc_b2)


def _score_body(zf_ref, f2_ref, cbt_ref, c2_ref, idx_ref, loss_ref):
    mm = jnp.dot(zf_ref[...], cbt_ref[...], preferred_element_type=jnp.float32)
    d = (f2_ref[...] + c2_ref[...]) - 2.0 * mm
    m = jnp.min(d, axis=1, keepdims=True)
    lane = lax.broadcasted_iota(jnp.int32, (FB, NUM_EMB), 1)
    ai = jnp.min(jnp.where(d == m, lane, NUM_EMB), axis=1)
    idx_ref[...] = ai.reshape(1, 1, FB)
    total = jnp.sum(m)

    @pl.when(pl.program_id(0) == 0)
    def _init():
        loss_ref[0, 0] = total

    @pl.when(pl.program_id(0) != 0)
    def _acc():
        loss_ref[0, 0] += total


def _run_score(zf, f2, cbt, c2):
    return pl.pallas_call(
        _score_body,
        grid=(FGRID,),
        in_specs=[
            pl.BlockSpec((FB, EMBED_DIM), lambda b: (b, 0)),
            pl.BlockSpec((FB, 1), lambda b: (b, 0)),
            pl.BlockSpec((EMBED_DIM, NUM_EMB), lambda b: (0, 0)),
            pl.BlockSpec((1, NUM_EMB), lambda b: (0, 0)),
        ],
        out_specs=[
            pl.BlockSpec((1, 1, FB), lambda b: (b, 0, 0)),
            pl.BlockSpec(block_shape=(1, 1), index_map=lambda b: (0, 0),
                         memory_space=pltpu.SMEM),
        ],
        out_shape=[
            jax.ShapeDtypeStruct((FGRID, 1, FB), jnp.int32),
            jax.ShapeDtypeStruct((1, 1), jnp.float32),
        ],
    )(zf, f2, cbt, c2)


def _sc_gather(codebook, idx_flat):
    mesh = plsc.VectorSubcoreMesh(core_axis_name="c", subcore_axis_name="s")

    @functools.partial(
        pl.kernel, mesh=mesh,
        compiler_params=pltpu.CompilerParams(use_tc_tiling_on_sc=False),
        out_type=jax.ShapeDtypeStruct((N_FLAT, EMBED_DIM), jnp.float32),
        scratch_types=[
            pltpu.VMEM((SC_CH,), jnp.int32),
            pltpu.VMEM((SC_CH, EMBED_DIM), jnp.float32),
            pltpu.SemaphoreType.DMA,
        ],
    )
    def k(idx_hbm, table_hbm, out_hbm, idx_v, rows_v, sem):
        wid = lax.axis_index("s") * SC_NC + lax.axis_index("c")
        for chunk in range(SC_PER_W // SC_CH):
            base = wid * SC_PER_W + chunk * SC_CH
            pltpu.sync_copy(idx_hbm.at[pl.ds(base, SC_CH)], idx_v)
            pltpu.async_copy(table_hbm.at[idx_v], rows_v, sem).wait()
            pltpu.sync_copy(rows_v, out_hbm.at[pl.ds(base, SC_CH)])

    return k(idx_flat, codebook)


def _dec_body(zq_ref, w1_ref, b1_ref, w2_ref, b2_ref, out_ref):
    hd = jnp.dot(zq_ref[...], w1_ref[...], preferred_element_type=jnp.float32)
    hd = jnp.maximum(hd + b1_ref[...], 0.0)
    out_ref[...] = jnp.dot(hd, w2_ref[...],
                           preferred_element_type=jnp.float32) + b2_ref[...]


def _run_dec(zq, dec_w1, dec_b1, dec_w2, dec_b2):
    return pl.pallas_call(
        _dec_body,
        grid=(GRID,),
        in_specs=[
            pl.BlockSpec((BB, NUM_LATENTS * EMBED_DIM), lambda b: (b, 0)),
            pl.BlockSpec((NUM_LATENTS * EMBED_DIM, HID), lambda b: (0, 0)),
            pl.BlockSpec((1, HID), lambda b: (0, 0)),
            pl.BlockSpec((HID, INPUT_DIM), lambda b: (0, 0)),
            pl.BlockSpec((1, INPUT_DIM), lambda b: (0, 0)),
        ],
        out_specs=pl.BlockSpec((BB, INPUT_DIM), lambda b: (b, 0)),
        out_shape=jax.ShapeDtypeStruct((B, INPUT_DIM), jnp.float32),
    )(zq, dec_w1, dec_b1, dec_w2, dec_b2)


def kernel(x, enc_w1, enc_b1, enc_w2, enc_b2, codebook,
           dec_w1, dec_b1, dec_w2, dec_b2):
    x = x.astype(jnp.float32)
    cbt = codebook.T
    c2 = jnp.sum(codebook ** 2, axis=1)[None, :]

    ze = _run_enc(x, enc_w1, enc_b1[None, :], enc_w2, enc_b2[None, :])
    zf = ze.reshape(N_FLAT, EMBED_DIM)
    f2 = jnp.sum(zf ** 2, axis=1, keepdims=True)

    idx3, loss_sum = _run_score(zf, f2, cbt, c2)
    idx_flat = idx3.reshape(N_FLAT)

    zq_flat = _sc_gather(codebook, idx_flat)
    zq = zq_flat.reshape(B, NUM_LATENTS * EMBED_DIM)

    x_recon = _run_dec(zq, dec_w1, dec_b1[None, :], dec_w2, dec_b2[None, :])

    vq_loss = (1.0 + CC) * loss_sum[0, 0] / jnp.float32(N_FLAT * EMBED_DIM)

    avg_probs = jnp.float32(1.0 / NUM_EMB)
    perplexity = jnp.exp(-(avg_probs * jnp.log(avg_probs + 1e-10)))

    encoding_indices = idx_flat.reshape(B, NUM_LATENTS, 1)
    return (x_recon, vq_loss, perplexity, encoding_indices)


# speedup vs baseline: 2.3280x; 1.0763x over previous
import functools

import jax
import jax.numpy as jnp
from jax import lax
from jax.experimental import pallas as pl
from jax.experimental.pallas import tpu as pltpu
from jax.experimental.pallas import tpu_sc as plsc

B = 4096
INPUT_DIM = 2048
NUM_LATENTS = 64
EMBED_DIM = 64
NUM_EMB = 1024
CC = 0.25
HID = 64
N_FLAT = B * NUM_LATENTS

BB = 512
GRID = B // BB
FB = 2048
FGRID = N_FLAT // FB

SC_NC = 2
SC_NS = 16
SC_NW = SC_NC * SC_NS
SC_PER_W = N_FLAT // SC_NW
SC_CH = 1024


def _enc_body(x_ref, w1_ref, b1_ref, w2_ref, b2_ref, zf_ref):
    h = jnp.dot(x_ref[...], w1_ref[...], preferred_element_type=jnp.float32)
    h = jnp.maximum(h + b1_ref[...], 0.0)
    ze = jnp.dot(h, w2_ref[...], preferred_element_type=jnp.float32)
    zf_ref[...] = ze + b2_ref[...]


def _run_enc(x, enc_w1, enc_b1, enc_w2, enc_b2):
    return pl.pallas_call(
        _enc_body,
        grid=(GRID,),
        in_specs=[
            pl.BlockSpec((BB, INPUT_DIM), lambda b: (b, 0)),
            pl.BlockSpec((INPUT_DIM, HID), lambda b: (0, 0)),
            pl.BlockSpec((1, HID), lambda b: (0, 0)),
            pl.BlockSpec((HID, NUM_LATENTS * EMBED_DIM), lambda b: (0, 0)),
            pl.BlockSpec((1, NUM_LATENTS * EMBED_DIM), lambda b: (0, 0)),
        ],
        out_specs=pl.BlockSpec((BB, NUM_LATENTS * EMBED_DIM), lambda b: (b, 0)),
        out_shape=jax.ShapeDtypeStruct((B, NUM_LATENTS * EMBED_DIM),
                                       jnp.float32),
    )(x, enc_w1, enc_b1, enc_w2, enc_b2)


RC = 256


def _score_body(zf_ref, f2_ref, cbt2_ref, c2_ref, lane_ref,
                idx_ref, loss_ref):
    cbt2 = cbt2_ref[...]
    c2 = c2_ref[...]
    lane = lane_ref[...]
    total = jnp.float32(0.0)
    for r in range(FB // RC):
        sl = pl.ds(r * RC, RC)
        mm2 = jnp.dot(zf_ref[sl, :], cbt2, preferred_element_type=jnp.float32)
        d = (f2_ref[sl, :] + c2) - mm2
        m = jnp.min(d, axis=1, keepdims=True)
        ai = jnp.min(jnp.where(d == m, lane, jnp.float32(NUM_EMB)),
                     axis=1).astype(jnp.int32)
        idx_ref[:, :, sl] = ai.reshape(1, 1, RC)
        total = total + jnp.sum(m)

    @pl.when(pl.program_id(0) == 0)
    def _init():
        loss_ref[0, 0] = total

    @pl.when(pl.program_id(0) != 0)
    def _acc():
        loss_ref[0, 0] += total


def _run_score(zf, f2, cbt, c2, lanef):
    return pl.pallas_call(
        _score_body,
        grid=(FGRID,),
        in_specs=[
            pl.BlockSpec((FB, EMBED_DIM), lambda b: (b, 0)),
            pl.BlockSpec((FB, 1), lambda b: (b, 0)),
            pl.BlockSpec((EMBED_DIM, NUM_EMB), lambda b: (0, 0)),
            pl.BlockSpec((1, NUM_EMB), lambda b: (0, 0)),
            pl.BlockSpec((1, NUM_EMB), lambda b: (0, 0)),
        ],
        out_specs=[
            pl.BlockSpec((1, 1, FB), lambda b: (b, 0, 0)),
            pl.BlockSpec(block_shape=(1, 1), index_map=lambda b: (0, 0),
                         memory_space=pltpu.SMEM),
        ],
        out_shape=[
            jax.ShapeDtypeStruct((FGRID, 1, FB), jnp.int32),
            jax.ShapeDtypeStruct((1, 1), jnp.float32),
        ],
    )(zf, f2, cbt, c2, lanef)


def _sc_gather(codebook, idx_flat):
    mesh = plsc.VectorSubcoreMesh(core_axis_name="c", subcore_axis_name="s")

    @functools.partial(
        pl.kernel, mesh=mesh,
        compiler_params=pltpu.CompilerParams(use_tc_tiling_on_sc=False),
        out_type=jax.ShapeDtypeStruct((N_FLAT, EMBED_DIM), jnp.float32),
        scratch_types=[
            pltpu.VMEM((SC_CH,), jnp.int32),
            pltpu.VMEM((SC_CH, EMBED_DIM), jnp.float32),
            pltpu.SemaphoreType.DMA,
        ],
    )
    def k(idx_hbm, table_hbm, out_hbm, idx_v, rows_v, sem):
        wid = lax.axis_index("s") * SC_NC + lax.axis_index("c")
        for chunk in range(SC_PER_W // SC_CH):
            base = wid * SC_PER_W + chunk * SC_CH
            pltpu.sync_copy(idx_hbm.at[pl.ds(base, SC_CH)], idx_v)
            pltpu.async_copy(table_hbm.at[idx_v], rows_v, sem).wait()
            pltpu.sync_copy(rows_v, out_hbm.at[pl.ds(base, SC_CH)])

    return k(idx_flat, codebook)


def _dec_body(zq_ref, w1_ref, b1_ref, w2_ref, b2_ref, out_ref):
    hd = jnp.dot(zq_ref[...], w1_ref[...], preferred_element_type=jnp.float32)
    hd = jnp.maximum(hd + b1_ref[...], 0.0)
    out_ref[...] = jnp.dot(hd, w2_ref[...],
                           preferred_element_type=jnp.float32) + b2_ref[...]


def _run_dec(zq, dec_w1, dec_b1, dec_w2, dec_b2):
    return pl.pallas_call(
        _dec_body,
        grid=(GRID,),
        in_specs=[
            pl.BlockSpec((BB, NUM_LATENTS * EMBED_DIM), lambda b: (b, 0)),
            pl.BlockSpec((NUM_LATENTS * EMBED_DIM, HID), lambda b: (0, 0)),
            pl.BlockSpec((1, HID), lambda b: (0, 0)),
            pl.BlockSpec((HID, INPUT_DIM), lambda b: (0, 0)),
            pl.BlockSpec((1, INPUT_DIM), lambda b: (0, 0)),
        ],
        out_specs=pl.BlockSpec((BB, INPUT_DIM), lambda b: (b, 0)),
        out_shape=jax.ShapeDtypeStruct((B, INPUT_DIM), jnp.float32),
    )(zq, dec_w1, dec_b1, dec_w2, dec_b2)


def kernel(x, enc_w1, enc_b1, enc_w2, enc_b2, codebook,
           dec_w1, dec_b1, dec_w2, dec_b2):
    x = x.astype(jnp.float32)
    cbt2 = 2.0 * codebook.T
    c2 = jnp.sum(codebook ** 2, axis=1)[None, :]

    ze = _run_enc(x, enc_w1, enc_b1[None, :], enc_w2, enc_b2[None, :])
    zf = ze.reshape(N_FLAT, EMBED_DIM)
    f2 = jnp.sum(zf ** 2, axis=1, keepdims=True)

    lanef = jnp.arange(NUM_EMB, dtype=jnp.float32)[None, :]
    idx3, loss_sum = _run_score(zf, f2, cbt2, c2, lanef)
    idx_flat = idx3.reshape(N_FLAT)

    zq_flat = _sc_gather(codebook, idx_flat)
    zq = zq_flat.reshape(B, NUM_LATENTS * EMBED_DIM)

    x_recon = _run_dec(zq, dec_w1, dec_b1[None, :], dec_w2, dec_b2[None, :])

    vq_loss = (1.0 + CC) * loss_sum[0, 0] / jnp.float32(N_FLAT * EMBED_DIM)

    avg_probs = jnp.float32(1.0 / NUM_EMB)
    perplexity = jnp.exp(-(avg_probs * jnp.log(avg_probs + 1e-10)))

    encoding_indices = idx_flat.reshape(B, NUM_LATENTS, 1)
    return (x_recon, vq_loss, perplexity, encoding_indices)


# speedup vs baseline: 2.4544x; 1.0543x over previous
import functools

import jax
import jax.numpy as jnp
from jax import lax
from jax.experimental import pallas as pl
from jax.experimental.pallas import tpu as pltpu
from jax.experimental.pallas import tpu_sc as plsc

B = 4096
INPUT_DIM = 2048
NUM_LATENTS = 64
EMBED_DIM = 64
NUM_EMB = 1024
CC = 0.25
HID = 64
N_FLAT = B * NUM_LATENTS

BB = 512
GRID = B // BB
FB = 2048
FGRID = N_FLAT // FB

SC_NC = 2
SC_NS = 16
SC_NW = SC_NC * SC_NS
SC_PER_W = N_FLAT // SC_NW
SC_CH = 1024


def _enc_body(x_ref, w1_ref, b1_ref, w2_ref, b2_ref, zf_ref):
    h = jnp.dot(x_ref[...], w1_ref[...], preferred_element_type=jnp.float32)
    h = jnp.maximum(h + b1_ref[...], 0.0)
    ze = jnp.dot(h, w2_ref[...], preferred_element_type=jnp.float32)
    zf_ref[...] = ze + b2_ref[...]


def _run_enc(x, enc_w1, enc_b1, enc_w2, enc_b2):
    return pl.pallas_call(
        _enc_body,
        grid=(GRID,),
        in_specs=[
            pl.BlockSpec((BB, INPUT_DIM), lambda b: (b, 0)),
            pl.BlockSpec((INPUT_DIM, HID), lambda b: (0, 0)),
            pl.BlockSpec((1, HID), lambda b: (0, 0)),
            pl.BlockSpec((HID, NUM_LATENTS * EMBED_DIM), lambda b: (0, 0)),
            pl.BlockSpec((1, NUM_LATENTS * EMBED_DIM), lambda b: (0, 0)),
        ],
        out_specs=pl.BlockSpec((BB, NUM_LATENTS * EMBED_DIM), lambda b: (b, 0)),
        out_shape=jax.ShapeDtypeStruct((B, NUM_LATENTS * EMBED_DIM),
                                       jnp.float32),
    )(x, enc_w1, enc_b1, enc_w2, enc_b2)


RC = 256


def _score_body(zf_ref, f2_ref, cbt2_ref, c2_ref, lane_ref,
                idx_ref, loss_ref):
    cbt2 = cbt2_ref[...]
    c2 = c2_ref[...]
    lane = lane_ref[...]
    total = jnp.float32(0.0)
    for r in range(FB // RC):
        sl = pl.ds(r * RC, RC)
        mm2 = jnp.dot(zf_ref[sl, :], cbt2, preferred_element_type=jnp.float32)
        d = (f2_ref[sl, :] + c2) - mm2
        m = jnp.min(d, axis=1, keepdims=True)
        ai = jnp.min(jnp.where(d == m, lane, jnp.float32(NUM_EMB)),
                     axis=1).astype(jnp.int32)
        idx_ref[:, :, sl] = ai.reshape(1, 1, RC)
        total = total + jnp.sum(m)

    @pl.when(pl.program_id(0) == 0)
    def _init():
        loss_ref[0, 0] = total

    @pl.when(pl.program_id(0) != 0)
    def _acc():
        loss_ref[0, 0] += total


def _run_score(zf, f2, cbt, c2, lanef):
    return pl.pallas_call(
        _score_body,
        grid=(FGRID,),
        in_specs=[
            pl.BlockSpec((FB, EMBED_DIM), lambda b: (b, 0)),
            pl.BlockSpec((FB, 1), lambda b: (b, 0)),
            pl.BlockSpec((EMBED_DIM, NUM_EMB), lambda b: (0, 0)),
            pl.BlockSpec((1, NUM_EMB), lambda b: (0, 0)),
            pl.BlockSpec((1, NUM_EMB), lambda b: (0, 0)),
        ],
        out_specs=[
            pl.BlockSpec((1, 1, FB), lambda b: (b, 0, 0)),
            pl.BlockSpec(block_shape=(1, 1), index_map=lambda b: (0, 0),
                         memory_space=pltpu.SMEM),
        ],
        out_shape=[
            jax.ShapeDtypeStruct((FGRID, 1, FB), jnp.int32),
            jax.ShapeDtypeStruct((1, 1), jnp.float32),
        ],
    )(zf, f2, cbt, c2, lanef)


def _tab_body(cb_ref, w1r_ref, t_ref):
    t_ref[...] = jnp.dot(cb_ref[...], w1r_ref[0],
                         preferred_element_type=jnp.float32)


def _run_tables(codebook, dec_w1r):
    return pl.pallas_call(
        _tab_body,
        grid=(NUM_LATENTS,),
        in_specs=[
            pl.BlockSpec((NUM_EMB, EMBED_DIM), lambda i: (0, 0)),
            pl.BlockSpec((1, EMBED_DIM, HID), lambda i: (i, 0, 0)),
        ],
        out_specs=pl.BlockSpec((NUM_EMB, HID), lambda i: (i, 0)),
        out_shape=jax.ShapeDtypeStruct((NUM_LATENTS * NUM_EMB, HID),
                                       jnp.float32),
    )(codebook, dec_w1r)


SC_BPW = B // SC_NW


def _sc_gather_add(table, idx_off, bloc, zeros):
    mesh = plsc.VectorSubcoreMesh(core_axis_name="c", subcore_axis_name="s")

    @functools.partial(
        pl.kernel, mesh=mesh,
        compiler_params=pltpu.CompilerParams(use_tc_tiling_on_sc=False),
        out_type=jax.ShapeDtypeStruct((B, HID), jnp.float32),
        scratch_types=[
            pltpu.VMEM((SC_CH,), jnp.int32),
            pltpu.VMEM((SC_CH,), jnp.int32),
            pltpu.VMEM((SC_CH, HID), jnp.float32),
            pltpu.VMEM_SHARED((B, HID), jnp.float32),
            pltpu.SemaphoreType.DMA,
        ],
    )
    def k(idx_hbm, table_hbm, bloc_hbm, zeros_hbm, out_hbm,
          idx_v, bloc_v, rows_v, hd_sh, sem):
        wid = lax.axis_index("s") * SC_NC + lax.axis_index("c")
        obase = wid * SC_BPW
        pltpu.sync_copy(zeros_hbm, hd_sh.at[pl.ds(obase, SC_BPW)])
        for chunk in range(SC_PER_W // SC_CH):
            base = wid * SC_PER_W + chunk * SC_CH
            pltpu.sync_copy(idx_hbm.at[pl.ds(base, SC_CH)], idx_v)
            pltpu.async_copy(table_hbm.at[idx_v], rows_v, sem).wait()
            pltpu.sync_copy(bloc_hbm.at[pl.ds(base, SC_CH)], bloc_v)
            pltpu.sync_copy(rows_v, hd_sh.at[bloc_v], add=True)
        pltpu.sync_copy(hd_sh.at[pl.ds(obase, SC_BPW)],
                        out_hbm.at[pl.ds(obase, SC_BPW)])

    return k(idx_off, table, bloc, zeros)


def _dec_body(hd_ref, b1_ref, w2_ref, b2_ref, out_ref):
    hd = jnp.maximum(hd_ref[...] + b1_ref[...], 0.0)
    out_ref[...] = jnp.dot(hd, w2_ref[...],
                           preferred_element_type=jnp.float32) + b2_ref[...]


def _run_dec(hd, dec_b1, dec_w2, dec_b2):
    return pl.pallas_call(
        _dec_body,
        grid=(GRID,),
        in_specs=[
            pl.BlockSpec((BB, HID), lambda b: (b, 0)),
            pl.BlockSpec((1, HID), lambda b: (0, 0)),
            pl.BlockSpec((HID, INPUT_DIM), lambda b: (0, 0)),
            pl.BlockSpec((1, INPUT_DIM), lambda b: (0, 0)),
        ],
        out_specs=pl.BlockSpec((BB, INPUT_DIM), lambda b: (b, 0)),
        out_shape=jax.ShapeDtypeStruct((B, INPUT_DIM), jnp.float32),
    )(hd, dec_b1, dec_w2, dec_b2)


def kernel(x, enc_w1, enc_b1, enc_w2, enc_b2, codebook,
           dec_w1, dec_b1, dec_w2, dec_b2):
    x = x.astype(jnp.float32)
    cbt2 = 2.0 * codebook.T
    c2 = jnp.sum(codebook ** 2, axis=1)[None, :]

    ze = _run_enc(x, enc_w1, enc_b1[None, :], enc_w2, enc_b2[None, :])
    zf = ze.reshape(N_FLAT, EMBED_DIM)
    f2 = jnp.sum(zf ** 2, axis=1, keepdims=True)

    lanef = jnp.arange(NUM_EMB, dtype=jnp.float32)[None, :]
    idx3, loss_sum = _run_score(zf, f2, cbt2, c2, lanef)
    idx_flat = idx3.reshape(N_FLAT)

    table = _run_tables(codebook, dec_w1.reshape(NUM_LATENTS, EMBED_DIM, HID))
    idx_off = idx_flat + (jnp.arange(N_FLAT, dtype=jnp.int32)
                          % NUM_LATENTS) * NUM_EMB
    bloc = jnp.arange(N_FLAT, dtype=jnp.int32) // NUM_LATENTS
    zeros = jnp.zeros((SC_BPW, HID), jnp.float32)
    hd = _sc_gather_add(table, idx_off, bloc, zeros)

    x_recon = _run_dec(hd, dec_b1[None, :], dec_w2, dec_b2[None, :])

    vq_loss = (1.0 + CC) * loss_sum[0, 0] / jnp.float32(N_FLAT * EMBED_DIM)

    avg_probs = jnp.float32(1.0 / NUM_EMB)
    perplexity = jnp.exp(-(avg_probs * jnp.log(avg_probs + 1e-10)))

    encoding_indices = idx_flat.reshape(B, NUM_LATENTS, 1)
    return (x_recon, vq_loss, perplexity, encoding_indices)


# speedup vs baseline: 2.4665x; 1.0050x over previous
import functools

import jax
import jax.numpy as jnp
from jax import lax
from jax.experimental import pallas as pl
from jax.experimental.pallas import tpu as pltpu
from jax.experimental.pallas import tpu_sc as plsc

B = 4096
INPUT_DIM = 2048
NUM_LATENTS = 64
EMBED_DIM = 64
NUM_EMB = 1024
CC = 0.25
HID = 64
N_FLAT = B * NUM_LATENTS

BB = 512
GRID = B // BB
FB = 2048
FGRID = N_FLAT // FB

SC_NC = 2
SC_NS = 16
SC_NW = SC_NC * SC_NS
SC_PER_W = N_FLAT // SC_NW
SC_CH = 512


def _enc_body(x_ref, w1_ref, b1_ref, w2_ref, b2_ref, zf_ref):
    h = jnp.dot(x_ref[...], w1_ref[...], preferred_element_type=jnp.float32)
    h = jnp.maximum(h + b1_ref[...], 0.0)
    ze = jnp.dot(h, w2_ref[...], preferred_element_type=jnp.float32)
    zf_ref[...] = ze + b2_ref[...]


def _run_enc(x, enc_w1, enc_b1, enc_w2, enc_b2):
    return pl.pallas_call(
        _enc_body,
        grid=(GRID,),
        in_specs=[
            pl.BlockSpec((BB, INPUT_DIM), lambda b: (b, 0)),
            pl.BlockSpec((INPUT_DIM, HID), lambda b: (0, 0)),
            pl.BlockSpec((1, HID), lambda b: (0, 0)),
            pl.BlockSpec((HID, NUM_LATENTS * EMBED_DIM), lambda b: (0, 0)),
            pl.BlockSpec((1, NUM_LATENTS * EMBED_DIM), lambda b: (0, 0)),
        ],
        out_specs=pl.BlockSpec((BB, NUM_LATENTS * EMBED_DIM), lambda b: (b, 0)),
        out_shape=jax.ShapeDtypeStruct((B, NUM_LATENTS * EMBED_DIM),
                                       jnp.float32),
    )(x, enc_w1, enc_b1, enc_w2, enc_b2)


RC = 256


def _score_body(zf_ref, f2_ref, cbt2_ref, c2_ref, lane_ref,
                idx_ref, loss_ref):
    cbt2 = cbt2_ref[...]
    c2 = c2_ref[...]
    lane = lane_ref[...]
    total = jnp.float32(0.0)
    for r in range(FB // RC):
        sl = pl.ds(r * RC, RC)
        mm2 = jnp.dot(zf_ref[sl, :], cbt2, preferred_element_type=jnp.float32)
        d = (f2_ref[sl, :] + c2) - mm2
        m = jnp.min(d, axis=1, keepdims=True)
        ai = jnp.min(jnp.where(d == m, lane, jnp.float32(NUM_EMB)),
                     axis=1).astype(jnp.int32)
        idx_ref[:, :, sl] = ai.reshape(1, 1, RC)
        total = total + jnp.sum(m)

    @pl.when(pl.program_id(0) == 0)
    def _init():
        loss_ref[0, 0] = total

    @pl.when(pl.program_id(0) != 0)
    def _acc():
        loss_ref[0, 0] += total


def _run_score(zf, f2, cbt, c2, lanef):
    return pl.pallas_call(
        _score_body,
        grid=(FGRID,),
        in_specs=[
            pl.BlockSpec((FB, EMBED_DIM), lambda b: (b, 0)),
            pl.BlockSpec((FB, 1), lambda b: (b, 0)),
            pl.BlockSpec((EMBED_DIM, NUM_EMB), lambda b: (0, 0)),
            pl.BlockSpec((1, NUM_EMB), lambda b: (0, 0)),
            pl.BlockSpec((1, NUM_EMB), lambda b: (0, 0)),
        ],
        out_specs=[
            pl.BlockSpec((1, 1, FB), lambda b: (b, 0, 0)),
            pl.BlockSpec(block_shape=(1, 1), index_map=lambda b: (0, 0),
                         memory_space=pltpu.SMEM),
        ],
        out_shape=[
            jax.ShapeDtypeStruct((FGRID, 1, FB), jnp.int32),
            jax.ShapeDtypeStruct((1, 1), jnp.float32),
        ],
    )(zf, f2, cbt, c2, lanef)


def _tab_body(cb_ref, w1r_ref, t_ref):
    t_ref[...] = jnp.dot(cb_ref[...], w1r_ref[0],
                         preferred_element_type=jnp.float32)


def _run_tables(codebook, dec_w1r):
    return pl.pallas_call(
        _tab_body,
        grid=(NUM_LATENTS,),
        in_specs=[
            pl.BlockSpec((NUM_EMB, EMBED_DIM), lambda i: (0, 0)),
            pl.BlockSpec((1, EMBED_DIM, HID), lambda i: (i, 0, 0)),
        ],
        out_specs=pl.BlockSpec((NUM_EMB, HID), lambda i: (i, 0)),
        out_shape=jax.ShapeDtypeStruct((NUM_LATENTS * NUM_EMB, HID),
                                       jnp.float32),
    )(codebook, dec_w1r)


SC_BPW = B // SC_NW


def _sc_gather_add(table, idx_off, bloc, zeros):
    mesh = plsc.VectorSubcoreMesh(core_axis_name="c", subcore_axis_name="s")

    @functools.partial(
        pl.kernel, mesh=mesh,
        compiler_params=pltpu.CompilerParams(use_tc_tiling_on_sc=False),
        out_type=jax.ShapeDtypeStruct((B, HID), jnp.float32),
        scratch_types=[
            pltpu.VMEM((2, SC_CH), jnp.int32),
            pltpu.VMEM((2, SC_CH), jnp.int32),
            pltpu.VMEM((2, SC_CH, HID), jnp.float32),
            pltpu.VMEM_SHARED((B, HID), jnp.float32),
            pltpu.SemaphoreType.DMA,
            pltpu.SemaphoreType.DMA,
        ],
    )
    def k(idx_hbm, table_hbm, bloc_hbm, zeros_hbm, out_hbm,
          idx_v, bloc_v, rows_v, hd_sh, sem0, sem1):
        wid = lax.axis_index("s") * SC_NC + lax.axis_index("c")
        obase = wid * SC_BPW
        nchunk = SC_PER_W // SC_CH
        sems = (sem0, sem1)
        pltpu.sync_copy(zeros_hbm, hd_sh.at[pl.ds(obase, SC_BPW)])

        def fire(chunk, buf):
            base = wid * SC_PER_W + chunk * SC_CH
            pltpu.sync_copy(idx_hbm.at[pl.ds(base, SC_CH)], idx_v.at[buf])
            pltpu.sync_copy(bloc_hbm.at[pl.ds(base, SC_CH)], bloc_v.at[buf])
            return pltpu.async_copy(table_hbm.at[idx_v.at[buf]],
                                    rows_v.at[buf], sems[buf])

        pending = fire(0, 0)
        for chunk in range(nchunk):
            buf = chunk % 2
            pending.wait()
            if chunk + 1 < nchunk:
                pending = fire(chunk + 1, 1 - buf)
            pltpu.sync_copy(rows_v.at[buf], hd_sh.at[bloc_v.at[buf]],
                            add=True)
        pltpu.sync_copy(hd_sh.at[pl.ds(obase, SC_BPW)],
                        out_hbm.at[pl.ds(obase, SC_BPW)])

    return k(idx_off, table, bloc, zeros)


def _dec_body(hd_ref, b1_ref, w2_ref, b2_ref, out_ref):
    hd = jnp.maximum(hd_ref[...] + b1_ref[...], 0.0)
    out_ref[...] = jnp.dot(hd, w2_ref[...],
                           preferred_element_type=jnp.float32) + b2_ref[...]


def _run_dec(hd, dec_b1, dec_w2, dec_b2):
    return pl.pallas_call(
        _dec_body,
        grid=(GRID,),
        in_specs=[
            pl.BlockSpec((BB, HID), lambda b: (b, 0)),
            pl.BlockSpec((1, HID), lambda b: (0, 0)),
            pl.BlockSpec((HID, INPUT_DIM), lambda b: (0, 0)),
            pl.BlockSpec((1, INPUT_DIM), lambda b: (0, 0)),
        ],
        out_specs=pl.BlockSpec((BB, INPUT_DIM), lambda b: (b, 0)),
        out_shape=jax.ShapeDtypeStruct((B, INPUT_DIM), jnp.float32),
    )(hd, dec_b1, dec_w2, dec_b2)


def kernel(x, enc_w1, enc_b1, enc_w2, enc_b2, codebook,
           dec_w1, dec_b1, dec_w2, dec_b2):
    x = x.astype(jnp.float32)
    cbt2 = 2.0 * codebook.T
    c2 = jnp.sum(codebook ** 2, axis=1)[None, :]

    ze = _run_enc(x, enc_w1, enc_b1[None, :], enc_w2, enc_b2[None, :])
    zf = ze.reshape(N_FLAT, EMBED_DIM)
    f2 = jnp.sum(zf ** 2, axis=1, keepdims=True)

    lanef = jnp.arange(NUM_EMB, dtype=jnp.float32)[None, :]
    idx3, loss_sum = _run_score(zf, f2, cbt2, c2, lanef)
    idx_flat = idx3.reshape(N_FLAT)

    table = _run_tables(codebook, dec_w1.reshape(NUM_LATENTS, EMBED_DIM, HID))
    idx_off = idx_flat + (jnp.arange(N_FLAT, dtype=jnp.int32)
                          % NUM_LATENTS) * NUM_EMB
    bloc = jnp.arange(N_FLAT, dtype=jnp.int32) // NUM_LATENTS
    zeros = jnp.zeros((SC_BPW, HID), jnp.float32)
    hd = _sc_gather_add(table, idx_off, bloc, zeros)

    x_recon = _run_dec(hd, dec_b1[None, :], dec_w2, dec_b2[None, :])

    vq_loss = (1.0 + CC) * loss_sum[0, 0] / jnp.float32(N_FLAT * EMBED_DIM)

    avg_probs = jnp.float32(1.0 / NUM_EMB)
    perplexity = jnp.exp(-(avg_probs * jnp.log(avg_probs + 1e-10)))

    encoding_indices = idx_flat.reshape(B, NUM_LATENTS, 1)
    return (x_recon, vq_loss, perplexity, encoding_indices)


# speedup vs baseline: 2.6118x; 1.0589x over previous
import functools

import jax
import jax.numpy as jnp
from jax import lax
from jax.experimental import pallas as pl
from jax.experimental.pallas import tpu as pltpu
from jax.experimental.pallas import tpu_sc as plsc

B = 4096
INPUT_DIM = 2048
NUM_LATENTS = 64
EMBED_DIM = 64
NUM_EMB = 1024
CC = 0.25
HID = 64
N_FLAT = B * NUM_LATENTS

BB = 512
GRID = B // BB
FB = 4096
FGRID = N_FLAT // FB

SC_NC = 2
SC_NS = 16
SC_NW = SC_NC * SC_NS
SC_PER_W = N_FLAT // SC_NW
SC_CH = 512


def _enc_body(x_ref, w1_ref, b1_ref, w2_ref, b2_ref, zf_ref):
    h = jnp.dot(x_ref[...], w1_ref[...], preferred_element_type=jnp.float32)
    h = jnp.maximum(h + b1_ref[...], 0.0)
    ze = jnp.dot(h, w2_ref[...], preferred_element_type=jnp.float32)
    zf_ref[...] = ze + b2_ref[...]


def _run_enc(x, enc_w1, enc_b1, enc_w2, enc_b2):
    return pl.pallas_call(
        _enc_body,
        grid=(GRID,),
        in_specs=[
            pl.BlockSpec((BB, INPUT_DIM), lambda b: (b, 0)),
            pl.BlockSpec((INPUT_DIM, HID), lambda b: (0, 0)),
            pl.BlockSpec((1, HID), lambda b: (0, 0)),
            pl.BlockSpec((HID, NUM_LATENTS * EMBED_DIM), lambda b: (0, 0)),
            pl.BlockSpec((1, NUM_LATENTS * EMBED_DIM), lambda b: (0, 0)),
        ],
        out_specs=pl.BlockSpec((BB, NUM_LATENTS * EMBED_DIM), lambda b: (b, 0)),
        out_shape=jax.ShapeDtypeStruct((B, NUM_LATENTS * EMBED_DIM),
                                       jnp.float32),
    )(x, enc_w1, enc_b1, enc_w2, enc_b2)


RC = 256


def _score_body(zf_ref, f2_ref, cbt2_ref, c2_ref, lane_ref,
                idx_ref, loss_ref):
    cbt2 = cbt2_ref[...]
    c2 = c2_ref[...]
    lane = lane_ref[...]
    total = jnp.float32(0.0)
    for r in range(FB // RC):
        sl = pl.ds(r * RC, RC)
        mm2 = jnp.dot(zf_ref[sl, :], cbt2, preferred_element_type=jnp.float32)
        d = (f2_ref[sl, :] + c2) - mm2
        m = jnp.min(d, axis=1, keepdims=True)
        ai = jnp.min(jnp.where(d == m, lane, jnp.float32(NUM_EMB)),
                     axis=1).astype(jnp.int32)
        idx_ref[:, :, sl] = ai.reshape(1, 1, RC)
        total = total + jnp.sum(m)

    @pl.when(pl.program_id(0) == 0)
    def _init():
        loss_ref[0, 0] = total

    @pl.when(pl.program_id(0) != 0)
    def _acc():
        loss_ref[0, 0] += total


def _run_score(zf, f2, cbt, c2, lanef):
    return pl.pallas_call(
        _score_body,
        grid=(FGRID,),
        in_specs=[
            pl.BlockSpec((FB, EMBED_DIM), lambda b: (b, 0)),
            pl.BlockSpec((FB, 1), lambda b: (b, 0)),
            pl.BlockSpec((EMBED_DIM, NUM_EMB), lambda b: (0, 0)),
            pl.BlockSpec((1, NUM_EMB), lambda b: (0, 0)),
            pl.BlockSpec((1, NUM_EMB), lambda b: (0, 0)),
        ],
        out_specs=[
            pl.BlockSpec((1, 1, FB), lambda b: (b, 0, 0)),
            pl.BlockSpec(block_shape=(1, 1), index_map=lambda b: (0, 0),
                         memory_space=pltpu.SMEM),
        ],
        out_shape=[
            jax.ShapeDtypeStruct((FGRID, 1, FB), jnp.int32),
            jax.ShapeDtypeStruct((1, 1), jnp.float32),
        ],
    )(zf, f2, cbt, c2, lanef)


def _tab_body(cb_ref, w1r_ref, t_ref):
    t_ref[...] = jnp.dot(cb_ref[...], w1r_ref[0],
                         preferred_element_type=jnp.float32)


def _run_tables(codebook, dec_w1r):
    return pl.pallas_call(
        _tab_body,
        grid=(NUM_LATENTS,),
        in_specs=[
            pl.BlockSpec((NUM_EMB, EMBED_DIM), lambda i: (0, 0)),
            pl.BlockSpec((1, EMBED_DIM, HID), lambda i: (i, 0, 0)),
        ],
        out_specs=pl.BlockSpec((NUM_EMB, HID), lambda i: (i, 0)),
        out_shape=jax.ShapeDtypeStruct((NUM_LATENTS * NUM_EMB, HID),
                                       jnp.float32),
    )(codebook, dec_w1r)


SC_BPW = B // SC_NW


def _sc_gather_add(table, idx_off, bloc, zeros):
    mesh = plsc.VectorSubcoreMesh(core_axis_name="c", subcore_axis_name="s")

    @functools.partial(
        pl.kernel, mesh=mesh,
        compiler_params=pltpu.CompilerParams(use_tc_tiling_on_sc=False),
        out_type=jax.ShapeDtypeStruct((B, HID), jnp.float32),
        scratch_types=[
            pltpu.VMEM((2, SC_CH), jnp.int32),
            pltpu.VMEM((2, SC_CH), jnp.int32),
            pltpu.VMEM((2, SC_CH, HID), jnp.float32),
            pltpu.VMEM_SHARED((B, HID), jnp.float32),
            pltpu.SemaphoreType.DMA,
            pltpu.SemaphoreType.DMA,
        ],
    )
    def k(idx_hbm, table_hbm, bloc_hbm, zeros_hbm, out_hbm,
          idx_v, bloc_v, rows_v, hd_sh, sem0, sem1):
        wid = lax.axis_index("s") * SC_NC + lax.axis_index("c")
        obase = wid * SC_BPW
        nchunk = SC_PER_W // SC_CH
        sems = (sem0, sem1)
        pltpu.sync_copy(zeros_hbm, hd_sh.at[pl.ds(obase, SC_BPW)])

        def fire(chunk, buf):
            base = wid * SC_PER_W + chunk * SC_CH
            pltpu.sync_copy(idx_hbm.at[pl.ds(base, SC_CH)], idx_v.at[buf])
            pltpu.sync_copy(bloc_hbm.at[pl.ds(base, SC_CH)], bloc_v.at[buf])
            return pltpu.async_copy(table_hbm.at[idx_v.at[buf]],
                                    rows_v.at[buf], sems[buf])

        pending = fire(0, 0)
        for chunk in range(nchunk):
            buf = chunk % 2
            pending.wait()
            if chunk + 1 < nchunk:
                pending = fire(chunk + 1, 1 - buf)
            pltpu.sync_copy(rows_v.at[buf], hd_sh.at[bloc_v.at[buf]],
                            add=True)
        pltpu.sync_copy(hd_sh.at[pl.ds(obase, SC_BPW)],
                        out_hbm.at[pl.ds(obase, SC_BPW)])

    return k(idx_off, table, bloc, zeros)


def _dec_body(hd_ref, b1_ref, w2_ref, b2_ref, out_ref):
    hd = jnp.maximum(hd_ref[...] + b1_ref[...], 0.0)
    out_ref[...] = jnp.dot(hd, w2_ref[...],
                           preferred_element_type=jnp.float32) + b2_ref[...]


def _run_dec(hd, dec_b1, dec_w2, dec_b2):
    return pl.pallas_call(
        _dec_body,
        grid=(GRID,),
        in_specs=[
            pl.BlockSpec((BB, HID), lambda b: (b, 0)),
            pl.BlockSpec((1, HID), lambda b: (0, 0)),
            pl.BlockSpec((HID, INPUT_DIM), lambda b: (0, 0)),
            pl.BlockSpec((1, INPUT_DIM), lambda b: (0, 0)),
        ],
        out_specs=pl.BlockSpec((BB, INPUT_DIM), lambda b: (b, 0)),
        out_shape=jax.ShapeDtypeStruct((B, INPUT_DIM), jnp.float32),
    )(hd, dec_b1, dec_w2, dec_b2)


def kernel(x, enc_w1, enc_b1, enc_w2, enc_b2, codebook,
           dec_w1, dec_b1, dec_w2, dec_b2):
    x = x.astype(jnp.float32)
    cbt2 = 2.0 * codebook.T
    c2 = jnp.sum(codebook ** 2, axis=1)[None, :]

    ze = _run_enc(x, enc_w1, enc_b1[None, :], enc_w2, enc_b2[None, :])
    zf = ze.reshape(N_FLAT, EMBED_DIM)
    f2 = jnp.sum(zf ** 2, axis=1, keepdims=True)

    lanef = jnp.arange(NUM_EMB, dtype=jnp.float32)[None, :]
    idx3, loss_sum = _run_score(zf, f2, cbt2, c2, lanef)
    idx_flat = idx3.reshape(N_FLAT)

    table = _run_tables(codebook, dec_w1.reshape(NUM_LATENTS, EMBED_DIM, HID))
    idx_off = idx_flat + (jnp.arange(N_FLAT, dtype=jnp.int32)
                          % NUM_LATENTS) * NUM_EMB
    bloc = jnp.arange(N_FLAT, dtype=jnp.int32) // NUM_LATENTS
    zeros = jnp.zeros((SC_BPW, HID), jnp.float32)
    hd = _sc_gather_add(table, idx_off, bloc, zeros)

    x_recon = _run_dec(hd, dec_b1[None, :], dec_w2, dec_b2[None, :])

    vq_loss = (1.0 + CC) * loss_sum[0, 0] / jnp.float32(N_FLAT * EMBED_DIM)

    avg_probs = jnp.float32(1.0 / NUM_EMB)
    perplexity = jnp.exp(-(avg_probs * jnp.log(avg_probs + 1e-10)))

    encoding_indices = idx_flat.reshape(B, NUM_LATENTS, 1)
    return (x_recon, vq_loss, perplexity, encoding_indices)


# speedup vs baseline: 2.6160x; 1.0016x over previous
import functools

import jax
import jax.numpy as jnp
from jax import lax
from jax.experimental import pallas as pl
from jax.experimental.pallas import tpu as pltpu
from jax.experimental.pallas import tpu_sc as plsc

B = 4096
INPUT_DIM = 2048
NUM_LATENTS = 64
EMBED_DIM = 64
NUM_EMB = 1024
CC = 0.25
HID = 64
N_FLAT = B * NUM_LATENTS

BB = 512
GRID = B // BB
FB = 4096
FGRID = N_FLAT // FB

SC_NC = 2
SC_NS = 16
SC_NW = SC_NC * SC_NS
SC_PER_W = N_FLAT // SC_NW
SC_CH = 512


def _enc_body(x_ref, w1_ref, b1_ref, w2_ref, b2_ref, zf_ref):
    h = jnp.dot(x_ref[...], w1_ref[...], preferred_element_type=jnp.float32)
    h = jnp.maximum(h + b1_ref[...], 0.0)
    ze = jnp.dot(h, w2_ref[...], preferred_element_type=jnp.float32)
    zf_ref[...] = ze + b2_ref[...]


def _run_enc(x, enc_w1, enc_b1, enc_w2, enc_b2):
    return pl.pallas_call(
        _enc_body,
        grid=(GRID,),
        in_specs=[
            pl.BlockSpec((BB, INPUT_DIM), lambda b: (b, 0)),
            pl.BlockSpec((INPUT_DIM, HID), lambda b: (0, 0)),
            pl.BlockSpec((1, HID), lambda b: (0, 0)),
            pl.BlockSpec((HID, NUM_LATENTS * EMBED_DIM), lambda b: (0, 0)),
            pl.BlockSpec((1, NUM_LATENTS * EMBED_DIM), lambda b: (0, 0)),
        ],
        out_specs=pl.BlockSpec((BB, NUM_LATENTS * EMBED_DIM), lambda b: (b, 0)),
        out_shape=jax.ShapeDtypeStruct((B, NUM_LATENTS * EMBED_DIM),
                                       jnp.float32),
    )(x, enc_w1, enc_b1, enc_w2, enc_b2)


RC = 256


def _score_body(zf_ref, f2_ref, cbt2_ref, c2_ref, lane_ref,
                idx_ref, loss_ref):
    cbt2 = cbt2_ref[...]
    c2 = c2_ref[...]
    lane = lane_ref[...]
    total = jnp.float32(0.0)
    for r in range(FB // RC):
        sl = pl.ds(r * RC, RC)
        mm2 = jnp.dot(zf_ref[sl, :], cbt2, preferred_element_type=jnp.float32)
        d = (f2_ref[sl, :] + c2) - mm2
        m = jnp.min(d, axis=1, keepdims=True)
        ai = jnp.min(jnp.where(d == m, lane, jnp.float32(NUM_EMB)),
                     axis=1).astype(jnp.int32)
        idx_ref[:, :, sl] = ai.reshape(1, 1, RC)
        total = total + jnp.sum(m)

    loss_ref[0, 0, 0] = total


def _run_score(zf, f2, cbt, c2, lanef):
    return pl.pallas_call(
        _score_body,
        grid=(FGRID,),
        in_specs=[
            pl.BlockSpec((FB, EMBED_DIM), lambda b: (b, 0)),
            pl.BlockSpec((FB, 1), lambda b: (b, 0)),
            pl.BlockSpec((EMBED_DIM, NUM_EMB), lambda b: (0, 0)),
            pl.BlockSpec((1, NUM_EMB), lambda b: (0, 0)),
            pl.BlockSpec((1, NUM_EMB), lambda b: (0, 0)),
        ],
        out_specs=[
            pl.BlockSpec((1, 1, FB), lambda b: (b, 0, 0)),
            pl.BlockSpec(block_shape=(1, 1, 1), index_map=lambda b: (b, 0, 0),
                         memory_space=pltpu.SMEM),
        ],
        out_shape=[
            jax.ShapeDtypeStruct((FGRID, 1, FB), jnp.int32),
            jax.ShapeDtypeStruct((FGRID, 1, 1), jnp.float32),
        ],
        compiler_params=pltpu.CompilerParams(
            dimension_semantics=("parallel",)),
    )(zf, f2, cbt, c2, lanef)


def _tab_body(cb_ref, w1r_ref, t_ref):
    t_ref[...] = jnp.dot(cb_ref[...], w1r_ref[0],
                         preferred_element_type=jnp.float32)


def _run_tables(codebook, dec_w1r):
    return pl.pallas_call(
        _tab_body,
        grid=(NUM_LATENTS,),
        in_specs=[
            pl.BlockSpec((NUM_EMB, EMBED_DIM), lambda i: (0, 0)),
            pl.BlockSpec((1, EMBED_DIM, HID), lambda i: (i, 0, 0)),
        ],
        out_specs=pl.BlockSpec((NUM_EMB, HID), lambda i: (i, 0)),
        out_shape=jax.ShapeDtypeStruct((NUM_LATENTS * NUM_EMB, HID),
                                       jnp.float32),
    )(codebook, dec_w1r)


SC_BPW = B // SC_NW


def _sc_gather_add(table, idx_off, bloc, zeros):
    mesh = plsc.VectorSubcoreMesh(core_axis_name="c", subcore_axis_name="s")

    @functools.partial(
        pl.kernel, mesh=mesh,
        compiler_params=pltpu.CompilerParams(use_tc_tiling_on_sc=False),
        out_type=jax.ShapeDtypeStruct((B, HID), jnp.float32),
        scratch_types=[
            pltpu.VMEM((2, SC_CH), jnp.int32),
            pltpu.VMEM((2, SC_CH), jnp.int32),
            pltpu.VMEM((2, SC_CH, HID), jnp.float32),
            pltpu.VMEM_SHARED((B, HID), jnp.float32),
            pltpu.SemaphoreType.DMA,
            pltpu.SemaphoreType.DMA,
        ],
    )
    def k(idx_hbm, table_hbm, bloc_hbm, zeros_hbm, out_hbm,
          idx_v, bloc_v, rows_v, hd_sh, sem0, sem1):
        wid = lax.axis_index("s") * SC_NC + lax.axis_index("c")
        obase = wid * SC_BPW
        nchunk = SC_PER_W // SC_CH
        sems = (sem0, sem1)
        pltpu.sync_copy(zeros_hbm, hd_sh.at[pl.ds(obase, SC_BPW)])

        def fire(chunk, buf):
            base = wid * SC_PER_W + chunk * SC_CH
            pltpu.sync_copy(idx_hbm.at[pl.ds(base, SC_CH)], idx_v.at[buf])
            pltpu.sync_copy(bloc_hbm.at[pl.ds(base, SC_CH)], bloc_v.at[buf])
            return pltpu.async_copy(table_hbm.at[idx_v.at[buf]],
                                    rows_v.at[buf], sems[buf])

        pending = fire(0, 0)
        for chunk in range(nchunk):
            buf = chunk % 2
            pending.wait()
            if chunk + 1 < nchunk:
                pending = fire(chunk + 1, 1 - buf)
            pltpu.sync_copy(rows_v.at[buf], hd_sh.at[bloc_v.at[buf]],
                            add=True)
        pltpu.sync_copy(hd_sh.at[pl.ds(obase, SC_BPW)],
                        out_hbm.at[pl.ds(obase, SC_BPW)])

    return k(idx_off, table, bloc, zeros)


def _dec_body(hd_ref, b1_ref, w2_ref, b2_ref, out_ref):
    hd = jnp.maximum(hd_ref[...] + b1_ref[...], 0.0)
    out_ref[...] = jnp.dot(hd, w2_ref[...],
                           preferred_element_type=jnp.float32) + b2_ref[...]


def _run_dec(hd, dec_b1, dec_w2, dec_b2):
    return pl.pallas_call(
        _dec_body,
        grid=(GRID,),
        in_specs=[
            pl.BlockSpec((BB, HID), lambda b: (b, 0)),
            pl.BlockSpec((1, HID), lambda b: (0, 0)),
            pl.BlockSpec((HID, INPUT_DIM), lambda b: (0, 0)),
            pl.BlockSpec((1, INPUT_DIM), lambda b: (0, 0)),
        ],
        out_specs=pl.BlockSpec((BB, INPUT_DIM), lambda b: (b, 0)),
        out_shape=jax.ShapeDtypeStruct((B, INPUT_DIM), jnp.float32),
    )(hd, dec_b1, dec_w2, dec_b2)


def kernel(x, enc_w1, enc_b1, enc_w2, enc_b2, codebook,
           dec_w1, dec_b1, dec_w2, dec_b2):
    x = x.astype(jnp.float32)
    cbt2 = 2.0 * codebook.T
    c2 = jnp.sum(codebook ** 2, axis=1)[None, :]

    ze = _run_enc(x, enc_w1, enc_b1[None, :], enc_w2, enc_b2[None, :])
    zf = ze.reshape(N_FLAT, EMBED_DIM)
    f2 = jnp.sum(zf ** 2, axis=1, keepdims=True)

    lanef = jnp.arange(NUM_EMB, dtype=jnp.float32)[None, :]
    idx3, loss_sum = _run_score(zf, f2, cbt2, c2, lanef)
    idx_flat = idx3.reshape(N_FLAT)

    table = _run_tables(codebook, dec_w1.reshape(NUM_LATENTS, EMBED_DIM, HID))
    idx_off = idx_flat + (jnp.arange(N_FLAT, dtype=jnp.int32)
                          % NUM_LATENTS) * NUM_EMB
    bloc = jnp.arange(N_FLAT, dtype=jnp.int32) // NUM_LATENTS
    zeros = jnp.zeros((SC_BPW, HID), jnp.float32)
    hd = _sc_gather_add(table, idx_off, bloc, zeros)

    x_recon = _run_dec(hd, dec_b1[None, :], dec_w2, dec_b2[None, :])

    vq_loss = ((1.0 + CC) * jnp.sum(loss_sum)
               / jnp.float32(N_FLAT * EMBED_DIM))

    avg_probs = jnp.float32(1.0 / NUM_EMB)
    perplexity = jnp.exp(-(avg_probs * jnp.log(avg_probs + 1e-10)))

    encoding_indices = idx_flat.reshape(B, NUM_LATENTS, 1)
    return (x_recon, vq_loss, perplexity, encoding_indices)
